# baseline (speedup 1.0000x reference)
import functools

import jax
import jax.numpy as jnp
import numpy as np
from jax import lax
from jax.experimental import pallas as pl
from jax.experimental.pallas import tpu as pltpu
from jax.experimental.pallas import tpu_sc as plsc

_BATCH, _HIST, _DIM = 16384, 20, 32
_NE = 1000000
_NC, _NS = 2, 16
_NW = _NC * _NS
_NCOL = _NE // 128
_CPW = 244
_NJ = 125000

_BLK = 256
_NTASK = _HIST * (_BATCH // _BLK)
_TPW = _NTASK // _NW

_mesh = plsc.VectorSubcoreMesh(core_axis_name="c", subcore_axis_name="s")


def _wid():
    return lax.axis_index("s") * _NC + lax.axis_index("c")


def _i16():
    return lax.iota(jnp.int32, 16)


_CBLK = 4
_NBLK = _CPW // _CBLK


@functools.partial(
    pl.kernel,
    mesh=_mesh,
    out_type=jax.ShapeDtypeStruct((_NJ, 128), jnp.float32),
    scratch_types=[
        pltpu.VMEM((32, 128 * _CBLK), jnp.float32),
        pltpu.VMEM((32, 128 * _CBLK), jnp.float32),
        pltpu.VMEM((16 * _CBLK, 128), jnp.float32),
        pltpu.VMEM((16 * _CBLK, 128), jnp.float32),
        pltpu.VMEM((32, 64), jnp.float32),
        pltpu.SemaphoreType.DMA,
        pltpu.SemaphoreType.DMA,
        pltpu.SemaphoreType.DMA,
        pltpu.SemaphoreType.DMA,
    ],
    compiler_params=pltpu.CompilerParams(needs_layout_passes=False),
)
def _detrans(tt_hbm, te_hbm, vin0, vin1, vout0, vout1, vtail,
             isem0, isem1, osem0, osem1):
    wid = _wid()
    start = wid * _NBLK
    it = _i16()
    src = [(it << 1) + (32 * g) for g in range(4)]
    jl = [(it >> 2) + (4 * g) for g in range(4)]
    dlb = (it & 3) << 5
    dl = [dlb + d for d in range(32)]

    def transpose_blk(vin, vout, ncol):
        for c in range(ncol):
            for d in range(32):
                dv = jnp.full((16,), d, jnp.int32)
                for g in range(4):
                    vals = plsc.load_gather(vin, [dv, src[g] + 128 * c])
                    plsc.store_scatter(vout, [jl[g] + 16 * c, dl[d]], vals)

    def in_slice(b):
        return tt_hbm.at[:, pl.ds(b * 128 * _CBLK, 128 * _CBLK)]

    def out_slice(b):
        return te_hbm.at[pl.ds(b * 16 * _CBLK, 16 * _CBLK)]

    pltpu.async_copy(in_slice(start), vin0, isem0)
    pltpu.async_copy(in_slice(start + 1), vin1, isem1)

    def pair(i2, carry):
        ba = start + 2 * i2
        for (b, vin, isem, vout, osem) in (
            (ba, vin0, isem0, vout0, osem0),
            (ba + 1, vin1, isem1, vout1, osem1),
        ):
            pltpu.make_async_copy(in_slice(b), vin, isem).wait()

            @pl.when(i2 > 0)
            def _():
                pltpu.make_async_copy(te_hbm.at[pl.ds(0, 16 * _CBLK)], vout,
                                      osem).wait()

            transpose_blk(vin, vout, _CBLK)
            pltpu.async_copy(vout, out_slice(b), osem)
            bn = jnp.minimum(b + 2, _NBLK * _NW - 1)
            pltpu.async_copy(in_slice(bn), vin, isem)
        return carry

    lax.fori_loop(0, (_NBLK - 1) // 2, pair, 0)
    pltpu.make_async_copy(in_slice(0), vin0, isem0).wait()
    pltpu.make_async_copy(in_slice(0), vin1, isem1).wait()
    pltpu.make_async_copy(te_hbm.at[pl.ds(0, 16 * _CBLK)], vout0, osem0).wait()
    pltpu.make_async_copy(te_hbm.at[pl.ds(0, 16 * _CBLK)], vout1, osem1).wait()
    blast = start + _NBLK - 1
    transpose_blk(vin0, vout0, _CBLK)
    pltpu.async_copy(vout0, out_slice(blast), osem0)
    pltpu.make_async_copy(te_hbm.at[pl.ds(0, 16 * _CBLK)], vout0, osem0).wait()

    @pl.when(wid < 4)
    def _():
        c = _CPW * _NW + wid
        csl = tt_hbm.at[:, pl.ds(c * 128, 128)]
        pltpu.async_copy(csl, vin0.at[:, pl.ds(0, 128)], isem0)
        pltpu.make_async_copy(csl, vin0.at[:, pl.ds(0, 128)], isem0).wait()
        transpose_blk(vin0, vout0, 1)
        pltpu.async_copy(vout0.at[pl.ds(0, 16)], te_hbm.at[pl.ds(c * 16, 16)],
                         osem0)
        pltpu.make_async_copy(te_hbm.at[pl.ds(0, 16)],
                              vout0.at[pl.ds(0, 16)], osem0).wait()

    @pl.when(wid == 31)
    def _():
        tsl = tt_hbm.at[:, pl.ds(_NCOL * 128, 64)]
        pltpu.async_copy(tsl, vtail, isem0)
        pltpu.make_async_copy(tsl, vtail, isem0).wait()

        for d in range(32):
            dv = jnp.full((16,), d, jnp.int32)
            for g in range(2):
                vals = plsc.load_gather(vtail, [dv, src[g]])
                plsc.store_scatter(vout0, [jl[g], dl[d]], vals)
        pltpu.async_copy(vout0.at[pl.ds(0, 8)],
                         te_hbm.at[pl.ds(_NCOL * 16, 8)], osem0)
        pltpu.make_async_copy(te_hbm.at[pl.ds(0, 8)], vout0.at[pl.ds(0, 8)],
                              osem0).wait()


@functools.partial(
    pl.kernel,
    mesh=_mesh,
    out_type=jax.ShapeDtypeStruct((_HIST, _DIM, _BATCH), jnp.float32),
    scratch_types=[
        pltpu.VMEM((_BLK,), jnp.int32),
        pltpu.VMEM((_BLK,), jnp.int32),
        pltpu.VMEM((2, 128), jnp.int32),
        pltpu.VMEM((2, 128), jnp.int32),
        pltpu.VMEM((_BLK,), jnp.int32),
        pltpu.VMEM((_BLK,), jnp.int32),
        pltpu.VMEM((_BLK, 128), jnp.float32),
        pltpu.VMEM((_BLK, 128), jnp.float32),
        pltpu.VMEM((_DIM, _BLK), jnp.float32),
        pltpu.VMEM((_DIM, _BLK), jnp.float32),
        pltpu.SemaphoreType.DMA,
        pltpu.SemaphoreType.DMA,
        pltpu.SemaphoreType.DMA,
        pltpu.SemaphoreType.DMA,
    ],
    compiler_params=pltpu.CompilerParams(needs_layout_passes=False),
)
def _gather(xt_hbm, te_hbm, out_hbm,
            ix0, ix1, ij0, ij1, xo0, xo1, rows0, rows1, dm0, dm1,
            gsem0, gsem1, osem0, osem1):
    wid = _wid()
    t0 = wid * _TPW
    it = _i16()
    li = [it + 16 * g for g in range(_BLK // 16)]

    ix = (ix0, ix1)
    ij = (ij0, ij1)
    xo = (xo0, xo1)
    rows = (rows0, rows1)
    dm = (dm0, dm1)
    gsem = (gsem0, gsem1)
    osem = (osem0, osem1)

    def task_hb(t):
        gt = t0 + t
        return gt // (_BATCH // _BLK), (gt % (_BATCH // _BLK)) * _BLK

    def prep(t, s):
        h, b0 = task_hb(t)
        pltpu.sync_copy(xt_hbm.at[h, pl.ds(b0, _BLK)], ix[s])

        def grp(k, c):
            v = ix[s][pl.ds(k * 16, 16)]
            ij[s][k // 8, pl.ds((k % 8) * 16, 16)] = v >> 2
            xo[s][pl.ds(k * 16, 16)] = (v & 3) * 32
            return c
        lax.fori_loop(0, _BLK // 16, grp, 0)
        for k in range(_BLK // 128):
            pltpu.async_copy(te_hbm.at[ij[s].at[k]],
                             rows[s].at[pl.ds(k * 128, 128)], gsem[s])

    def extract_and_out(t, s, first):
        h, b0 = task_hb(t)
        pltpu.make_async_copy(te_hbm.at[pl.ds(0, _BLK)], rows[s],
                              gsem[s]).wait()

        @pl.when(jnp.logical_not(first))
        def _():
            pltpu.make_async_copy(out_hbm.at[0, :, pl.ds(0, _BLK)], dm[s],
                                  osem[s]).wait()

        for g in range(_BLK // 16):
            xog = xo[s][pl.ds(16 * g, 16)]
            for d in range(32):
                vals = plsc.load_gather(rows[s], [li[g], xog + d])
                dm[s][d, pl.ds(16 * g, 16)] = vals
        pltpu.async_copy(dm[s], out_hbm.at[h, :, pl.ds(b0, _BLK)], osem[s])

    prep(0, 0)

    def pair(i2, carry):
        prep(2 * i2 + 1, 1)
        extract_and_out(2 * i2, 0, i2 == 0)

        @pl.when(i2 < _TPW // 2 - 1)
        def _():
            prep(2 * i2 + 2, 0)
        extract_and_out(2 * i2 + 1, 1, i2 == 0)
        return carry

    lax.fori_loop(0, _TPW // 2, pair, 0)
    for s in range(2):
        pltpu.make_async_copy(out_hbm.at[0, :, pl.ds(0, _BLK)], dm[s],
                              osem[s]).wait()


def kernel(x, table):
    xt = x.astype(jnp.int32).T
    tt = table.T
    te = _detrans(tt)
    out3 = _gather(xt, te)
    return out3.transpose(2, 0, 1)


# speedup vs baseline: 1.0168x; 1.0168x over previous
import functools

import jax
import jax.numpy as jnp
from jax import lax
from jax.experimental import pallas as pl
from jax.experimental.pallas import tpu as pltpu
from jax.experimental.pallas import tpu_sc as plsc

_BATCH, _HIST, _DIM = 16384, 20, 32
_NE = 1000000
_NC, _NS = 2, 16
_NW = _NC * _NS
_NCOL = _NE // 128
_CPW = 244
_NJ = 125000

_BLK = 128
_NTASK = _HIST * (_BATCH // _BLK)
_TPW = _NTASK // _NW

_mesh = plsc.VectorSubcoreMesh(core_axis_name="c", subcore_axis_name="s")


def _wid():
    return lax.axis_index("s") * _NC + lax.axis_index("c")


def _i16():
    return lax.iota(jnp.int32, 16)


@functools.partial(
    pl.kernel,
    mesh=_mesh,
    out_type=jax.ShapeDtypeStruct((_NJ, 128), jnp.float32),
    scratch_types=[
        pltpu.VMEM((32, 128), jnp.float32),
        pltpu.VMEM((32, 128), jnp.float32),
        pltpu.VMEM((32, 128), jnp.float32),
        pltpu.VMEM((32, 128), jnp.float32),
        pltpu.VMEM((16, 128), jnp.float32),
        pltpu.VMEM((16, 128), jnp.float32),
        pltpu.VMEM((32, 64), jnp.float32),
        pltpu.SemaphoreType.DMA,
        pltpu.SemaphoreType.DMA,
        pltpu.SemaphoreType.DMA,
        pltpu.SemaphoreType.DMA,
        pltpu.SemaphoreType.DMA,
        pltpu.SemaphoreType.DMA,
    ],
    compiler_params=pltpu.CompilerParams(needs_layout_passes=False),
)
def _detrans(tt_hbm, te_hbm, vin0, vin1, vin2, vin3, vout0, vout1, vtail,
             isem0, isem1, isem2, isem3, osem0, osem1):
    wid = _wid()
    start = wid * _CPW
    it = _i16()
    src = [(it << 1) + (32 * g) for g in range(4)]
    jl = [(it >> 2) + (4 * g) for g in range(4)]
    dlb = (it & 3) << 5
    dl = [dlb + d for d in range(32)]

    vins = (vin0, vin1, vin2, vin3)
    isems = (isem0, isem1, isem2, isem3)
    vouts = (vout0, vout1)
    osems = (osem0, osem1)

    def transpose_col(vin, vout, ngrp=4):
        for d in range(32):
            dv = jnp.full((16,), d, jnp.int32)
            vals = [plsc.load_gather(vin, [dv, src[g]]) for g in range(ngrp)]
            for g in range(ngrp):
                plsc.store_scatter(vout, [jl[g], dl[d]], vals[g])

    def in_slice(c):
        return tt_hbm.at[:, pl.ds(c * 128, 128)]

    def out_slice(c):
        return te_hbm.at[pl.ds(c * 16, 16)]

    for p in range(4):
        pltpu.async_copy(in_slice(start + p), vins[p], isems[p])

    def quad(i4, carry):
        c0 = start + 4 * i4
        for q in range(4):
            c = c0 + q
            vin, isem = vins[q], isems[q]
            vout, osem = vouts[q % 2], osems[q % 2]
            pltpu.make_async_copy(in_slice(c), vin, isem).wait()

            @pl.when(4 * i4 + q >= 2)
            def _():
                pltpu.make_async_copy(te_hbm.at[pl.ds(0, 16)], vout,
                                      osem).wait()

            transpose_col(vin, vout)
            pltpu.async_copy(vout, out_slice(c), osem)
            cn = jnp.minimum(c + 4, _CPW * _NW - 1)
            pltpu.async_copy(in_slice(cn), vin, isem)
        return carry

    lax.fori_loop(0, _CPW // 4, quad, 0)
    for p in range(4):
        pltpu.make_async_copy(in_slice(0), vins[p], isems[p]).wait()
    for p in range(2):
        pltpu.make_async_copy(te_hbm.at[pl.ds(0, 16)], vouts[p],
                              osems[p]).wait()

    @pl.when(wid < 4)
    def _():
        c = _CPW * _NW + wid
        pltpu.async_copy(in_slice(c), vin0, isem0)
        pltpu.make_async_copy(in_slice(c), vin0, isem0).wait()
        transpose_col(vin0, vout0)
        pltpu.async_copy(vout0, out_slice(c), osem0)
        pltpu.make_async_copy(te_hbm.at[pl.ds(0, 16)], vout0, osem0).wait()

    @pl.when(wid == 31)
    def _():
        tsl = tt_hbm.at[:, pl.ds(_NCOL * 128, 64)]
        pltpu.async_copy(tsl, vtail, isem0)
        pltpu.make_async_copy(tsl, vtail, isem0).wait()
        transpose_col(vtail, vout0, ngrp=2)
        pltpu.async_copy(vout0.at[pl.ds(0, 8)],
                         te_hbm.at[pl.ds(_NCOL * 16, 8)], osem0)
        pltpu.make_async_copy(te_hbm.at[pl.ds(0, 8)], vout0.at[pl.ds(0, 8)],
                              osem0).wait()


@functools.partial(
    pl.kernel,
    mesh=_mesh,
    out_type=jax.ShapeDtypeStruct((_HIST, _DIM, _BATCH), jnp.float32),
    scratch_types=[
        pltpu.VMEM((_BLK,), jnp.int32),
        pltpu.VMEM((_BLK,), jnp.int32),
        pltpu.VMEM((_BLK,), jnp.int32),
        pltpu.VMEM((_BLK,), jnp.int32),
        pltpu.VMEM((1, _BLK), jnp.int32),
        pltpu.VMEM((1, _BLK), jnp.int32),
        pltpu.VMEM((1, _BLK), jnp.int32),
        pltpu.VMEM((1, _BLK), jnp.int32),
        pltpu.VMEM((_BLK,), jnp.int32),
        pltpu.VMEM((_BLK,), jnp.int32),
        pltpu.VMEM((_BLK,), jnp.int32),
        pltpu.VMEM((_BLK,), jnp.int32),
        pltpu.VMEM((_BLK, 128), jnp.float32),
        pltpu.VMEM((_BLK, 128), jnp.float32),
        pltpu.VMEM((_BLK, 128), jnp.float32),
        pltpu.VMEM((_BLK, 128), jnp.float32),
        pltpu.VMEM((_DIM, _BLK), jnp.float32),
        pltpu.VMEM((_DIM, _BLK), jnp.float32),
        pltpu.SemaphoreType.DMA,
        pltpu.SemaphoreType.DMA,
        pltpu.SemaphoreType.DMA,
        pltpu.SemaphoreType.DMA,
        pltpu.SemaphoreType.DMA,
        pltpu.SemaphoreType.DMA,
        pltpu.SemaphoreType.DMA,
        pltpu.SemaphoreType.DMA,
        pltpu.SemaphoreType.DMA,
        pltpu.SemaphoreType.DMA,
    ],
    compiler_params=pltpu.CompilerParams(needs_layout_passes=False),
)
def _gather(xt_hbm, te_hbm, out_hbm,
            ix0, ix1, ix2, ix3, ij0, ij1, ij2, ij3,
            xo0, xo1, xo2, xo3, rows0, rows1, rows2, rows3, dm0, dm1,
            xsem0, xsem1, xsem2, xsem3,
            gsem0, gsem1, gsem2, gsem3, osem0, osem1):
    wid = _wid()
    t0 = wid * _TPW
    it = _i16()
    li = [it + 16 * g for g in range(_BLK // 16)]

    ix = (ix0, ix1, ix2, ix3)
    ij = (ij0, ij1, ij2, ij3)
    xo = (xo0, xo1, xo2, xo3)
    rows = (rows0, rows1, rows2, rows3)
    dm = (dm0, dm1)
    xsem = (xsem0, xsem1, xsem2, xsem3)
    gsem = (gsem0, gsem1, gsem2, gsem3)
    osem = (osem0, osem1)

    def task_hb(t):
        gt = t0 + t
        return gt // (_BATCH // _BLK), (gt % (_BATCH // _BLK)) * _BLK

    def idx_slice(t):
        h, b0 = task_hb(t)
        return xt_hbm.at[h, pl.ds(b0, _BLK)]

    def issue_idx(t, s):
        pltpu.async_copy(idx_slice(t), ix[s], xsem[s])

    def prep(t, s):
        pltpu.make_async_copy(idx_slice(t), ix[s], xsem[s]).wait()
        for k in range(_BLK // 16):
            v = ix[s][pl.ds(k * 16, 16)]
            ij[s][0, pl.ds(k * 16, 16)] = v >> 2
            xo[s][pl.ds(k * 16, 16)] = (v & 3) << 5
        pltpu.async_copy(te_hbm.at[ij[s].at[0]], rows[s], gsem[s])

    def extract_and_out(t, s, sd, first):
        h, b0 = task_hb(t)
        pltpu.make_async_copy(te_hbm.at[pl.ds(0, _BLK)], rows[s],
                              gsem[s]).wait()

        @pl.when(jnp.logical_not(first))
        def _():
            pltpu.make_async_copy(out_hbm.at[0, :, pl.ds(0, _BLK)], dm[sd],
                                  osem[sd]).wait()

        for g in range(_BLK // 16):
            xog = xo[s][pl.ds(16 * g, 16)]
            for d in range(32):
                vals = plsc.load_gather(rows[s], [li[g], xog + d])
                dm[sd][d, pl.ds(16 * g, 16)] = vals
        pltpu.async_copy(dm[sd], out_hbm.at[h, :, pl.ds(b0, _BLK)], osem[sd])

    for p in range(4):
        issue_idx(p, p)
    for p in range(3):
        prep(p, p)

    def quad(i4, carry):
        tq = 4 * i4
        for q in range(4):
            t = tq + q

            @pl.when(t + 4 < _TPW)
            def _():
                issue_idx(t + 4, q)

            @pl.when(t + 3 < _TPW)
            def _():
                prep(t + 3, (q + 3) % 4)
            extract_and_out(t, q, q % 2, t < 2)
        return carry

    lax.fori_loop(0, _TPW // 4, quad, 0)
    for sd in range(2):
        pltpu.make_async_copy(out_hbm.at[0, :, pl.ds(0, _BLK)], dm[sd],
                              osem[sd]).wait()


def kernel(x, table):
    xt = x.astype(jnp.int32).T
    tt = table.T
    te = _detrans(tt)
    out3 = _gather(xt, te)
    return out3.transpose(2, 0, 1)


# speedup vs baseline: 1.4685x; 1.4442x over previous
import functools

import jax
import jax.numpy as jnp
from jax import lax
from jax.experimental import pallas as pl
from jax.experimental.pallas import tpu as pltpu
from jax.experimental.pallas import tpu_sc as plsc

_BATCH, _HIST, _DIM = 16384, 20, 32
_NE = 1000000
_NC, _NS = 2, 16
_NW = _NC * _NS
_NCOL = _NE // 128
_CPW = 244
_NJ = 125000

_BLK = 128
_NTASK = _HIST * (_BATCH // _BLK)
_TPW = _NTASK // _NW

_mesh = plsc.VectorSubcoreMesh(core_axis_name="c", subcore_axis_name="s")


def _wid():
    return lax.axis_index("s") * _NC + lax.axis_index("c")


def _i16():
    return lax.iota(jnp.int32, 16)


@functools.partial(
    pl.kernel,
    mesh=_mesh,
    out_type=jax.ShapeDtypeStruct((_NJ, 128), jnp.float32),
    scratch_types=[
        pltpu.VMEM((32, 129), jnp.float32),
        pltpu.VMEM((32, 129), jnp.float32),
        pltpu.VMEM((32, 129), jnp.float32),
        pltpu.VMEM((32, 129), jnp.float32),
        pltpu.VMEM((16, 128), jnp.float32),
        pltpu.VMEM((16, 128), jnp.float32),
        pltpu.VMEM((32, 64), jnp.float32),
        pltpu.SemaphoreType.DMA,
        pltpu.SemaphoreType.DMA,
        pltpu.SemaphoreType.DMA,
        pltpu.SemaphoreType.DMA,
        pltpu.SemaphoreType.DMA,
        pltpu.SemaphoreType.DMA,
    ],
    compiler_params=pltpu.CompilerParams(needs_layout_passes=False),
)
def _detrans(tt_hbm, te_hbm, vin0, vin1, vin2, vin3, vout0, vout1, vtail,
             isem0, isem1, isem2, isem3, osem0, osem1):
    wid = _wid()
    start = wid * _CPW
    it = _i16()
    dlo = it
    dhi = it + 16

    vins = (vin0, vin1, vin2, vin3)
    isems = (isem0, isem1, isem2, isem3)
    vouts = (vout0, vout1)
    osems = (osem0, osem1)

    def transpose_col(vin, vout, nel=64):
        for el2 in range(nel):
            el = 2 * el2
            ev = jnp.full((16,), el, jnp.int32)
            v0 = plsc.load_gather(vin, [dlo, ev])
            v1 = plsc.load_gather(vin, [dhi, ev])
            jloc = el // 8
            lane = 32 * ((el % 8) // 2)
            vout[jloc, pl.ds(lane, 16)] = v0
            vout[jloc, pl.ds(lane + 16, 16)] = v1

    def in_slice(c):
        return tt_hbm.at[:, pl.ds(c * 128, 128)]

    def out_slice(c):
        return te_hbm.at[pl.ds(c * 16, 16)]

    for p in range(4):
        pltpu.async_copy(in_slice(start + p), vins[p].at[:, pl.ds(0, 128)],
                         isems[p])

    def quad(i4, carry):
        c0 = start + 4 * i4
        for q in range(4):
            c = c0 + q
            vin, isem = vins[q], isems[q]
            vout, osem = vouts[q % 2], osems[q % 2]
            pltpu.make_async_copy(in_slice(c), vin.at[:, pl.ds(0, 128)],
                                  isem).wait()

            @pl.when(4 * i4 + q >= 2)
            def _():
                pltpu.make_async_copy(te_hbm.at[pl.ds(0, 16)], vout,
                                      osem).wait()

            transpose_col(vin, vout)
            pltpu.async_copy(vout, out_slice(c), osem)
            cn = jnp.minimum(c + 4, _CPW * _NW - 1)
            pltpu.async_copy(in_slice(cn), vin.at[:, pl.ds(0, 128)], isem)
        return carry

    lax.fori_loop(0, _CPW // 4, quad, 0)
    for p in range(4):
        pltpu.make_async_copy(in_slice(0), vins[p].at[:, pl.ds(0, 128)],
                              isems[p]).wait()
    for p in range(2):
        pltpu.make_async_copy(te_hbm.at[pl.ds(0, 16)], vouts[p],
                              osems[p]).wait()

    @pl.when(wid < 4)
    def _():
        c = _CPW * _NW + wid
        pltpu.async_copy(in_slice(c), vin0.at[:, pl.ds(0, 128)], isem0)
        pltpu.make_async_copy(in_slice(c), vin0.at[:, pl.ds(0, 128)],
                              isem0).wait()
        transpose_col(vin0, vout0)
        pltpu.async_copy(vout0, out_slice(c), osem0)
        pltpu.make_async_copy(te_hbm.at[pl.ds(0, 16)], vout0, osem0).wait()

    @pl.when(wid == 31)
    def _():
        tsl = tt_hbm.at[:, pl.ds(_NCOL * 128, 64)]
        pltpu.async_copy(tsl, vtail, isem0)
        pltpu.make_async_copy(tsl, vtail, isem0).wait()
        transpose_col(vtail, vout0, nel=32)
        pltpu.async_copy(vout0.at[pl.ds(0, 8)],
                         te_hbm.at[pl.ds(_NCOL * 16, 8)], osem0)
        pltpu.make_async_copy(te_hbm.at[pl.ds(0, 8)], vout0.at[pl.ds(0, 8)],
                              osem0).wait()


@functools.partial(
    pl.kernel,
    mesh=_mesh,
    out_type=jax.ShapeDtypeStruct((_HIST, _DIM, _BATCH), jnp.float32),
    scratch_types=[
        pltpu.VMEM((_BLK,), jnp.int32),
        pltpu.VMEM((_BLK,), jnp.int32),
        pltpu.VMEM((_BLK,), jnp.int32),
        pltpu.VMEM((_BLK,), jnp.int32),
        pltpu.VMEM((1, _BLK), jnp.int32),
        pltpu.VMEM((1, _BLK), jnp.int32),
        pltpu.VMEM((1, _BLK), jnp.int32),
        pltpu.VMEM((1, _BLK), jnp.int32),
        pltpu.VMEM((_BLK,), jnp.int32),
        pltpu.VMEM((_BLK,), jnp.int32),
        pltpu.VMEM((_BLK,), jnp.int32),
        pltpu.VMEM((_BLK,), jnp.int32),
        pltpu.VMEM((_BLK, 128), jnp.float32),
        pltpu.VMEM((_BLK, 128), jnp.float32),
        pltpu.VMEM((_BLK, 128), jnp.float32),
        pltpu.VMEM((_BLK, 128), jnp.float32),
        pltpu.VMEM((_DIM, _BLK), jnp.float32),
        pltpu.VMEM((_DIM, _BLK), jnp.float32),
        pltpu.SemaphoreType.DMA,
        pltpu.SemaphoreType.DMA,
        pltpu.SemaphoreType.DMA,
        pltpu.SemaphoreType.DMA,
        pltpu.SemaphoreType.DMA,
        pltpu.SemaphoreType.DMA,
        pltpu.SemaphoreType.DMA,
        pltpu.SemaphoreType.DMA,
        pltpu.SemaphoreType.DMA,
        pltpu.SemaphoreType.DMA,
    ],
    compiler_params=pltpu.CompilerParams(needs_layout_passes=False),
)
def _gather(xt_hbm, te_hbm, out_hbm,
            ix0, ix1, ix2, ix3, ij0, ij1, ij2, ij3,
            xo0, xo1, xo2, xo3, rows0, rows1, rows2, rows3, dm0, dm1,
            xsem0, xsem1, xsem2, xsem3,
            gsem0, gsem1, gsem2, gsem3, osem0, osem1):
    wid = _wid()
    t0 = wid * _TPW
    it = _i16()
    li = [it + 16 * g for g in range(_BLK // 16)]
    dds = [(d0 + it) & 31 for d0 in range(32)]

    ix = (ix0, ix1, ix2, ix3)
    ij = (ij0, ij1, ij2, ij3)
    xo = (xo0, xo1, xo2, xo3)
    rows = (rows0, rows1, rows2, rows3)
    dm = (dm0, dm1)
    xsem = (xsem0, xsem1, xsem2, xsem3)
    gsem = (gsem0, gsem1, gsem2, gsem3)
    osem = (osem0, osem1)

    def task_hb(t):
        gt = t0 + t
        return gt // (_BATCH // _BLK), (gt % (_BATCH // _BLK)) * _BLK

    def idx_slice(t):
        h, b0 = task_hb(t)
        return xt_hbm.at[h, pl.ds(b0, _BLK)]

    def issue_idx(t, s):
        pltpu.async_copy(idx_slice(t), ix[s], xsem[s])

    def prep(t, s):
        pltpu.make_async_copy(idx_slice(t), ix[s], xsem[s]).wait()
        for k in range(_BLK // 16):
            v = ix[s][pl.ds(k * 16, 16)]
            ij[s][0, pl.ds(k * 16, 16)] = v >> 2
            xo[s][pl.ds(k * 16, 16)] = (v & 3) << 5
        pltpu.async_copy(te_hbm.at[ij[s].at[0]], rows[s], gsem[s])

    def extract_and_out(t, s, sd, first):
        h, b0 = task_hb(t)
        pltpu.make_async_copy(te_hbm.at[pl.ds(0, _BLK)], rows[s],
                              gsem[s]).wait()

        @pl.when(jnp.logical_not(first))
        def _():
            pltpu.make_async_copy(out_hbm.at[0, :, pl.ds(0, _BLK)], dm[sd],
                                  osem[sd]).wait()

        def grp(g, c):
            xog = xo[s][pl.ds(g * 16, 16)]
            lig = it + (g << 4)
            for d0 in range(32):
                vals = plsc.load_gather(rows[s], [lig, xog + dds[d0]])
                plsc.store_scatter(dm[sd], [dds[d0], lig], vals)
            return c
        lax.fori_loop(0, _BLK // 16, grp, 0)
        pltpu.async_copy(dm[sd], out_hbm.at[h, :, pl.ds(b0, _BLK)], osem[sd])

    for p in range(4):
        issue_idx(p, p)
    for p in range(3):
        prep(p, p)

    def quad(i4, carry):
        tq = 4 * i4
        for q in range(4):
            t = tq + q

            @pl.when(t + 4 < _TPW)
            def _():
                issue_idx(t + 4, q)

            @pl.when(t + 3 < _TPW)
            def _():
                prep(t + 3, (q + 3) % 4)
            extract_and_out(t, q, q % 2, t < 2)
        return carry

    lax.fori_loop(0, _TPW // 4, quad, 0)
    for sd in range(2):
        pltpu.make_async_copy(out_hbm.at[0, :, pl.ds(0, _BLK)], dm[sd],
                              osem[sd]).wait()


def kernel(x, table):
    xt = x.astype(jnp.int32).T
    tt = table.T
    te = _detrans(tt)
    out3 = _gather(xt, te)
    return out3.transpose(2, 0, 1)


# speedup vs baseline: 1.9229x; 1.3094x over previous
import functools

import jax
import jax.numpy as jnp
from jax import lax
from jax.experimental import pallas as pl
from jax.experimental.pallas import tpu as pltpu
from jax.experimental.pallas import tpu_sc as plsc

_BATCH, _HIST, _DIM = 16384, 20, 32
_NE = 1000000
_NC, _NS = 2, 16
_NW = _NC * _NS
_NCOL = _NE // 128
_CPW = 244
_NJ = 125000

_BLK = 128
_NTASK = _HIST * (_BATCH // _BLK)
_TPW = _NTASK // _NW

_mesh = plsc.VectorSubcoreMesh(core_axis_name="c", subcore_axis_name="s")


def _wid():
    return lax.axis_index("s") * _NC + lax.axis_index("c")


def _i16():
    return lax.iota(jnp.int32, 16)


@functools.partial(
    pl.kernel,
    mesh=_mesh,
    out_type=jax.ShapeDtypeStruct((_NJ, 128), jnp.float32),
    scratch_types=[
        pltpu.VMEM((32, 128), jnp.float32),
        pltpu.VMEM((32, 128), jnp.float32),
        pltpu.VMEM((32, 128), jnp.float32),
        pltpu.VMEM((32, 128), jnp.float32),
        pltpu.VMEM((16, 128), jnp.float32),
        pltpu.VMEM((16, 128), jnp.float32),
        pltpu.VMEM((32, 64), jnp.float32),
        pltpu.SemaphoreType.DMA,
        pltpu.SemaphoreType.DMA,
        pltpu.SemaphoreType.DMA,
        pltpu.SemaphoreType.DMA,
        pltpu.SemaphoreType.DMA,
        pltpu.SemaphoreType.DMA,
    ],
    compiler_params=pltpu.CompilerParams(needs_layout_passes=False),
)
def _detrans(tt_hbm, te_hbm, vin0, vin1, vin2, vin3, vout0, vout1, vtail,
             isem0, isem1, isem2, isem3, osem0, osem1):
    wid = _wid()
    start = wid * _CPW
    it = _i16()
    dds = [(p + it) & 31 for p in range(32)]
    elv = [(it << 1) + 32 * g2 for g2 in range(4)]
    jlv = [(it >> 2) + 4 * g2 for g2 in range(4)]
    lnv = [((it & 3) << 5) + dds[p] for p in range(32)]

    vins = (vin0, vin1, vin2, vin3)
    isems = (isem0, isem1, isem2, isem3)
    vouts = (vout0, vout1)
    osems = (osem0, osem1)

    def transpose_col(vin, vout, ngrp=4):
        for g2 in range(ngrp):
            for p in range(32):
                vals = plsc.load_gather(vin, [dds[p], elv[g2]])
                plsc.store_scatter(vout, [jlv[g2], lnv[p]], vals)

    def in_slice(c):
        return tt_hbm.at[:, pl.ds(c * 128, 128)]

    def out_slice(c):
        return te_hbm.at[pl.ds(c * 16, 16)]

    for p in range(4):
        pltpu.async_copy(in_slice(start + p), vins[p], isems[p])

    def quad(i4, carry):
        c0 = start + 4 * i4
        for q in range(4):
            c = c0 + q
            vin, isem = vins[q], isems[q]
            vout, osem = vouts[q % 2], osems[q % 2]
            pltpu.make_async_copy(in_slice(c), vin, isem).wait()

            @pl.when(4 * i4 + q >= 2)
            def _():
                pltpu.make_async_copy(te_hbm.at[pl.ds(0, 16)], vout,
                                      osem).wait()

            transpose_col(vin, vout)
            pltpu.async_copy(vout, out_slice(c), osem)
            cn = jnp.minimum(c + 4, _CPW * _NW - 1)
            pltpu.async_copy(in_slice(cn), vin, isem)
        return carry

    lax.fori_loop(0, _CPW // 4, quad, 0)
    for p in range(4):
        pltpu.make_async_copy(in_slice(0), vins[p], isems[p]).wait()
    for p in range(2):
        pltpu.make_async_copy(te_hbm.at[pl.ds(0, 16)], vouts[p],
                              osems[p]).wait()

    @pl.when(wid < 4)
    def _():
        c = _CPW * _NW + wid
        pltpu.async_copy(in_slice(c), vin0, isem0)
        pltpu.make_async_copy(in_slice(c), vin0, isem0).wait()
        transpose_col(vin0, vout0)
        pltpu.async_copy(vout0, out_slice(c), osem0)
        pltpu.make_async_copy(te_hbm.at[pl.ds(0, 16)], vout0, osem0).wait()

    @pl.when(wid == 31)
    def _():
        tsl = tt_hbm.at[:, pl.ds(_NCOL * 128, 64)]
        pltpu.async_copy(tsl, vtail, isem0)
        pltpu.make_async_copy(tsl, vtail, isem0).wait()
        transpose_col(vtail, vout0, ngrp=2)
        pltpu.async_copy(vout0.at[pl.ds(0, 8)],
                         te_hbm.at[pl.ds(_NCOL * 16, 8)], osem0)
        pltpu.make_async_copy(te_hbm.at[pl.ds(0, 8)], vout0.at[pl.ds(0, 8)],
                              osem0).wait()


@functools.partial(
    pl.kernel,
    mesh=_mesh,
    out_type=jax.ShapeDtypeStruct((_HIST, _DIM, _BATCH), jnp.float32),
    scratch_types=[
        pltpu.VMEM((_BLK,), jnp.int32),
        pltpu.VMEM((_BLK,), jnp.int32),
        pltpu.VMEM((_BLK,), jnp.int32),
        pltpu.VMEM((_BLK,), jnp.int32),
        pltpu.VMEM((1, _BLK), jnp.int32),
        pltpu.VMEM((1, _BLK), jnp.int32),
        pltpu.VMEM((1, _BLK), jnp.int32),
        pltpu.VMEM((1, _BLK), jnp.int32),
        pltpu.VMEM((_BLK,), jnp.int32),
        pltpu.VMEM((_BLK,), jnp.int32),
        pltpu.VMEM((_BLK,), jnp.int32),
        pltpu.VMEM((_BLK,), jnp.int32),
        pltpu.VMEM((_BLK, 128), jnp.float32),
        pltpu.VMEM((_BLK, 128), jnp.float32),
        pltpu.VMEM((_BLK, 128), jnp.float32),
        pltpu.VMEM((_BLK, 128), jnp.float32),
        pltpu.VMEM((_DIM, _BLK), jnp.float32),
        pltpu.VMEM((_DIM, _BLK), jnp.float32),
        pltpu.SemaphoreType.DMA,
        pltpu.SemaphoreType.DMA,
        pltpu.SemaphoreType.DMA,
        pltpu.SemaphoreType.DMA,
        pltpu.SemaphoreType.DMA,
        pltpu.SemaphoreType.DMA,
        pltpu.SemaphoreType.DMA,
        pltpu.SemaphoreType.DMA,
        pltpu.SemaphoreType.DMA,
        pltpu.SemaphoreType.DMA,
    ],
    compiler_params=pltpu.CompilerParams(needs_layout_passes=False),
)
def _gather(xt_hbm, te_hbm, out_hbm,
            ix0, ix1, ix2, ix3, ij0, ij1, ij2, ij3,
            xo0, xo1, xo2, xo3, rows0, rows1, rows2, rows3, dm0, dm1,
            xsem0, xsem1, xsem2, xsem3,
            gsem0, gsem1, gsem2, gsem3, osem0, osem1):
    wid = _wid()
    t0 = wid * _TPW
    it = _i16()
    li = [it + 16 * g for g in range(_BLK // 16)]
    dds = [(d0 + it) & 31 for d0 in range(32)]

    ix = (ix0, ix1, ix2, ix3)
    ij = (ij0, ij1, ij2, ij3)
    xo = (xo0, xo1, xo2, xo3)
    rows = (rows0, rows1, rows2, rows3)
    dm = (dm0, dm1)
    xsem = (xsem0, xsem1, xsem2, xsem3)
    gsem = (gsem0, gsem1, gsem2, gsem3)
    osem = (osem0, osem1)

    def task_hb(t):
        gt = t0 + t
        return gt // (_BATCH // _BLK), (gt % (_BATCH // _BLK)) * _BLK

    def idx_slice(t):
        h, b0 = task_hb(t)
        return xt_hbm.at[h, pl.ds(b0, _BLK)]

    def issue_idx(t, s):
        pltpu.async_copy(idx_slice(t), ix[s], xsem[s])

    def prep(t, s):
        pltpu.make_async_copy(idx_slice(t), ix[s], xsem[s]).wait()
        for k in range(_BLK // 16):
            v = ix[s][pl.ds(k * 16, 16)]
            ij[s][0, pl.ds(k * 16, 16)] = v >> 2
            xo[s][pl.ds(k * 16, 16)] = (v & 3) << 5
        pltpu.async_copy(te_hbm.at[ij[s].at[0]], rows[s], gsem[s])

    def extract_and_out(t, s, sd, first):
        h, b0 = task_hb(t)
        pltpu.make_async_copy(te_hbm.at[pl.ds(0, _BLK)], rows[s],
                              gsem[s]).wait()

        @pl.when(jnp.logical_not(first))
        def _():
            pltpu.make_async_copy(out_hbm.at[0, :, pl.ds(0, _BLK)], dm[sd],
                                  osem[sd]).wait()

        def grp(g, c):
            xog = xo[s][pl.ds(g * 16, 16)]
            lig = it + (g << 4)
            for d0 in range(32):
                vals = plsc.load_gather(rows[s], [lig, xog + dds[d0]])
                plsc.store_scatter(dm[sd], [dds[d0], lig], vals)
            return c
        lax.fori_loop(0, _BLK // 16, grp, 0)
        pltpu.async_copy(dm[sd], out_hbm.at[h, :, pl.ds(b0, _BLK)], osem[sd])

    for p in range(4):
        issue_idx(p, p)
    for p in range(3):
        prep(p, p)

    def quad(i4, carry):
        tq = 4 * i4
        for q in range(4):
            t = tq + q

            @pl.when(t + 4 < _TPW)
            def _():
                issue_idx(t + 4, q)

            @pl.when(t + 3 < _TPW)
            def _():
                prep(t + 3, (q + 3) % 4)
            extract_and_out(t, q, q % 2, t < 2)
        return carry

    lax.fori_loop(0, _TPW // 4, quad, 0)
    for sd in range(2):
        pltpu.make_async_copy(out_hbm.at[0, :, pl.ds(0, _BLK)], dm[sd],
                              osem[sd]).wait()


def kernel(x, table):
    xt = x.astype(jnp.int32).T
    tt = table.T
    te = _detrans(tt)
    out3 = _gather(xt, te)
    return out3.transpose(2, 0, 1)


# speedup vs baseline: 2.7692x; 1.4401x over previous
import functools

import jax
import jax.numpy as jnp
from jax import lax
from jax.experimental import pallas as pl
from jax.experimental.pallas import tpu as pltpu
from jax.experimental.pallas import tpu_sc as plsc

_BATCH, _HIST, _DIM = 16384, 20, 32
_NE = 1000000
_NC, _NS = 2, 16
_NW = _NC * _NS
_NCOL = _NE // 128
_CPW = 244
_NJ = 125000

_BLK = 128
_NTASK = _HIST * (_BATCH // _BLK)
_TPW = _NTASK // _NW

_mesh = plsc.VectorSubcoreMesh(core_axis_name="c", subcore_axis_name="s")


def _wid():
    return lax.axis_index("s") * _NC + lax.axis_index("c")


def _i16():
    return lax.iota(jnp.int32, 16)


@functools.partial(
    pl.kernel,
    mesh=_mesh,
    out_type=jax.ShapeDtypeStruct((_NJ, 128), jnp.float32),
    scratch_types=[
        pltpu.VMEM((32, 128), jnp.float32),
        pltpu.VMEM((32, 128), jnp.float32),
        pltpu.VMEM((32, 128), jnp.float32),
        pltpu.VMEM((32, 128), jnp.float32),
        pltpu.VMEM((16, 128), jnp.float32),
        pltpu.VMEM((16, 128), jnp.float32),
        pltpu.VMEM((32, 64), jnp.float32),
        pltpu.SemaphoreType.DMA,
        pltpu.SemaphoreType.DMA,
        pltpu.SemaphoreType.DMA,
        pltpu.SemaphoreType.DMA,
        pltpu.SemaphoreType.DMA,
        pltpu.SemaphoreType.DMA,
    ],
    compiler_params=pltpu.CompilerParams(needs_layout_passes=False),
)
def _detrans(tt_hbm, te_hbm, vin0, vin1, vin2, vin3, vout0, vout1, vtail,
             isem0, isem1, isem2, isem3, osem0, osem1):
    wid = _wid()
    start = wid * _CPW
    it = _i16()
    dds = [(p + it) & 31 for p in range(32)]
    elv = [(it << 1) + 32 * g2 for g2 in range(4)]
    jlv = [(it >> 2) + 4 * g2 for g2 in range(4)]
    lnv = [((it & 3) << 5) + dds[p] for p in range(32)]

    vins = (vin0, vin1, vin2, vin3)
    isems = (isem0, isem1, isem2, isem3)
    vouts = (vout0, vout1)
    osems = (osem0, osem1)

    def transpose_col(vin, vout, ngrp=4):
        def grp(g2, c):
            ev = (it << 1) + (g2 << 5)
            jv = (it >> 2) + (g2 << 2)
            for p in range(32):
                vals = plsc.load_gather(vin, [dds[p], ev])
                plsc.store_scatter(vout, [jv, lnv[p]], vals)
            return c
        lax.fori_loop(0, ngrp, grp, 0)

    def in_slice(c):
        return tt_hbm.at[:, pl.ds(c * 128, 128)]

    def out_slice(c):
        return te_hbm.at[pl.ds(c * 16, 16)]

    for p in range(4):
        pltpu.async_copy(in_slice(start + p), vins[p], isems[p])

    def quad(i4, carry):
        c0 = start + 4 * i4
        for q in range(4):
            c = c0 + q
            vin, isem = vins[q], isems[q]
            vout, osem = vouts[q % 2], osems[q % 2]
            pltpu.make_async_copy(in_slice(c), vin, isem).wait()

            @pl.when(4 * i4 + q >= 2)
            def _():
                pltpu.make_async_copy(te_hbm.at[pl.ds(0, 16)], vout,
                                      osem).wait()

            transpose_col(vin, vout)
            pltpu.async_copy(vout, out_slice(c), osem)
            cn = jnp.minimum(c + 4, _CPW * _NW - 1)
            pltpu.async_copy(in_slice(cn), vin, isem)
        return carry

    lax.fori_loop(0, _CPW // 4, quad, 0)
    for p in range(4):
        pltpu.make_async_copy(in_slice(0), vins[p], isems[p]).wait()
    for p in range(2):
        pltpu.make_async_copy(te_hbm.at[pl.ds(0, 16)], vouts[p],
                              osems[p]).wait()

    @pl.when(wid < 4)
    def _():
        c = _CPW * _NW + wid
        pltpu.async_copy(in_slice(c), vin0, isem0)
        pltpu.make_async_copy(in_slice(c), vin0, isem0).wait()
        transpose_col(vin0, vout0)
        pltpu.async_copy(vout0, out_slice(c), osem0)
        pltpu.make_async_copy(te_hbm.at[pl.ds(0, 16)], vout0, osem0).wait()

    @pl.when(wid == 31)
    def _():
        tsl = tt_hbm.at[:, pl.ds(_NCOL * 128, 64)]
        pltpu.async_copy(tsl, vtail, isem0)
        pltpu.make_async_copy(tsl, vtail, isem0).wait()
        transpose_col(vtail, vout0, ngrp=2)
        pltpu.async_copy(vout0.at[pl.ds(0, 8)],
                         te_hbm.at[pl.ds(_NCOL * 16, 8)], osem0)
        pltpu.make_async_copy(te_hbm.at[pl.ds(0, 8)], vout0.at[pl.ds(0, 8)],
                              osem0).wait()


@functools.partial(
    pl.kernel,
    mesh=_mesh,
    out_type=jax.ShapeDtypeStruct((_HIST, _DIM, _BATCH), jnp.float32),
    scratch_types=[
        pltpu.VMEM((_BLK,), jnp.int32),
        pltpu.VMEM((_BLK,), jnp.int32),
        pltpu.VMEM((_BLK,), jnp.int32),
        pltpu.VMEM((_BLK,), jnp.int32),
        pltpu.VMEM((1, _BLK), jnp.int32),
        pltpu.VMEM((1, _BLK), jnp.int32),
        pltpu.VMEM((1, _BLK), jnp.int32),
        pltpu.VMEM((1, _BLK), jnp.int32),
        pltpu.VMEM((_BLK,), jnp.int32),
        pltpu.VMEM((_BLK,), jnp.int32),
        pltpu.VMEM((_BLK,), jnp.int32),
        pltpu.VMEM((_BLK,), jnp.int32),
        pltpu.VMEM((_BLK, 128), jnp.float32),
        pltpu.VMEM((_BLK, 128), jnp.float32),
        pltpu.VMEM((_BLK, 128), jnp.float32),
        pltpu.VMEM((_BLK, 128), jnp.float32),
        pltpu.VMEM((_DIM, _BLK), jnp.float32),
        pltpu.VMEM((_DIM, _BLK), jnp.float32),
        pltpu.SemaphoreType.DMA,
        pltpu.SemaphoreType.DMA,
        pltpu.SemaphoreType.DMA,
        pltpu.SemaphoreType.DMA,
        pltpu.SemaphoreType.DMA,
        pltpu.SemaphoreType.DMA,
        pltpu.SemaphoreType.DMA,
        pltpu.SemaphoreType.DMA,
        pltpu.SemaphoreType.DMA,
        pltpu.SemaphoreType.DMA,
    ],
    compiler_params=pltpu.CompilerParams(needs_layout_passes=False),
)
def _gather(xt_hbm, te_hbm, out_hbm,
            ix0, ix1, ix2, ix3, ij0, ij1, ij2, ij3,
            xo0, xo1, xo2, xo3, rows0, rows1, rows2, rows3, dm0, dm1,
            xsem0, xsem1, xsem2, xsem3,
            gsem0, gsem1, gsem2, gsem3, osem0, osem1):
    wid = _wid()
    t0 = wid * _TPW
    it = _i16()
    li = [it + 16 * g for g in range(_BLK // 16)]
    dds = [(d0 + it) & 31 for d0 in range(32)]

    ix = (ix0, ix1, ix2, ix3)
    ij = (ij0, ij1, ij2, ij3)
    xo = (xo0, xo1, xo2, xo3)
    rows = (rows0, rows1, rows2, rows3)
    dm = (dm0, dm1)
    xsem = (xsem0, xsem1, xsem2, xsem3)
    gsem = (gsem0, gsem1, gsem2, gsem3)
    osem = (osem0, osem1)

    def task_hb(t):
        gt = t0 + t
        return gt // (_BATCH // _BLK), (gt % (_BATCH // _BLK)) * _BLK

    def idx_slice(t):
        h, b0 = task_hb(t)
        return xt_hbm.at[h, pl.ds(b0, _BLK)]

    def issue_idx(t, s):
        pltpu.async_copy(idx_slice(t), ix[s], xsem[s])

    def prep(t, s):
        pltpu.make_async_copy(idx_slice(t), ix[s], xsem[s]).wait()
        for k in range(_BLK // 16):
            v = ix[s][pl.ds(k * 16, 16)]
            ij[s][0, pl.ds(k * 16, 16)] = v >> 2
            xo[s][pl.ds(k * 16, 16)] = (v & 3) << 5
        pltpu.async_copy(te_hbm.at[ij[s].at[0]], rows[s], gsem[s])

    def extract_and_out(t, s, sd, first):
        h, b0 = task_hb(t)
        pltpu.make_async_copy(te_hbm.at[pl.ds(0, _BLK)], rows[s],
                              gsem[s]).wait()

        @pl.when(jnp.logical_not(first))
        def _():
            pltpu.make_async_copy(out_hbm.at[0, :, pl.ds(0, _BLK)], dm[sd],
                                  osem[sd]).wait()

        def grp(g, c):
            xog = xo[s][pl.ds(g * 16, 16)]
            lig = it + (g << 4)
            for d0 in range(32):
                vals = plsc.load_gather(rows[s], [lig, xog + dds[d0]])
                plsc.store_scatter(dm[sd], [dds[d0], lig], vals)
            return c
        lax.fori_loop(0, _BLK // 16, grp, 0)
        pltpu.async_copy(dm[sd], out_hbm.at[h, :, pl.ds(b0, _BLK)], osem[sd])

    for p in range(4):
        issue_idx(p, p)
    for p in range(3):
        prep(p, p)

    def quad(i4, carry):
        tq = 4 * i4
        for q in range(4):
            t = tq + q

            @pl.when(t + 4 < _TPW)
            def _():
                issue_idx(t + 4, q)

            @pl.when(t + 3 < _TPW)
            def _():
                prep(t + 3, (q + 3) % 4)
            extract_and_out(t, q, q % 2, t < 2)
        return carry

    lax.fori_loop(0, _TPW // 4, quad, 0)
    for sd in range(2):
        pltpu.make_async_copy(out_hbm.at[0, :, pl.ds(0, _BLK)], dm[sd],
                              osem[sd]).wait()


def kernel(x, table):
    xt = x.astype(jnp.int32).T
    tt = table.T
    te = _detrans(tt)
    out3 = _gather(xt, te)
    return out3.transpose(2, 0, 1)
